# baseline (speedup 1.0000x reference)
import jax
import jax.numpy as jnp
import numpy as np
from jax.experimental import pallas as pl

N_EXP = 64
D_MODEL = 8
D_FF = 16
BLK = 1024


def _moe_block(x_ref, M1_ref, Wall_ref, S2_ref, K_ref, Rx_ref, Rh_ref, o_ref):
    x = x_ref[...]
    t1 = jnp.dot(x, M1_ref[...], preferred_element_type=jnp.float32)
    logits = t1[:, :N_EXP]
    gs = t1[:, N_EXP:N_EXP + D_FF]
    us = t1[:, N_EXP + D_FF:N_EXP + 2 * D_FF]

    m = jnp.max(logits, axis=1, keepdims=True)
    w = 1.0 / jnp.sum(jnp.exp(logits - m), axis=1, keepdims=True)
    iota = jax.lax.broadcasted_iota(jnp.int32, logits.shape, 1)
    masked = jnp.where(logits == m, iota, N_EXP)
    first = jnp.min(masked, axis=1, keepdims=True)
    oh = (iota == first).astype(jnp.float32)

    Wt = jnp.dot(oh, Wall_ref[...], preferred_element_type=jnp.float32)

    xt = jnp.dot(x, Rx_ref[...], preferred_element_type=jnp.float32)
    pg = Wt[:, :128] * xt
    pu = Wt[:, 128:256] * xt
    gu = jnp.dot(jnp.concatenate([pg, pu], axis=1), S2_ref[...],
                 preferred_element_type=jnp.float32)
    g = gu[:, :D_FF]
    u = gu[:, D_FF:]
    h = (g * jax.nn.sigmoid(g)) * u * w

    ht = jnp.dot(h, Rh_ref[...], preferred_element_type=jnp.float32)
    pd = Wt[:, 256:384] * ht
    hs = (gs * jax.nn.sigmoid(gs)) * us
    o_ref[...] = jnp.dot(jnp.concatenate([pd, hs], axis=1), K_ref[...],
                         preferred_element_type=jnp.float32)


def kernel(hidden_states, gate_weight, Wg, Wu, Wd, Wsg, Wsu, Wsd):
    Bsz, S, D = hidden_states.shape
    T = Bsz * S
    x2 = hidden_states.reshape(T, D)

    M1 = jnp.concatenate([gate_weight.T, Wsg.T, Wsu.T], axis=1)
    Wall = jnp.concatenate(
        [Wg.reshape(N_EXP, 128), Wu.reshape(N_EXP, 128),
         Wd.reshape(N_EXP, 128)], axis=1)
    S8 = np.kron(np.eye(D_FF, dtype=np.float32), np.ones((D_MODEL, 1), np.float32))
    S16 = np.kron(np.eye(D_MODEL, dtype=np.float32), np.ones((D_FF, 1), np.float32))
    S2 = np.zeros((256, 2 * D_FF), np.float32)
    S2[:128, :D_FF] = S8
    S2[128:, D_FF:] = S8
    S2 = jnp.asarray(S2)
    K = jnp.concatenate([jnp.asarray(S16), Wsd.T], axis=0)
    Rx = jnp.asarray(np.kron(np.ones((1, D_FF), np.float32),
                             np.eye(D_MODEL, dtype=np.float32)))
    Rh = jnp.asarray(np.kron(np.ones((1, D_MODEL), np.float32),
                             np.eye(D_FF, dtype=np.float32)))

    full = lambda arr: pl.BlockSpec(arr.shape, lambda i: (0, 0))
    out = pl.pallas_call(
        _moe_block,
        grid=(T // BLK,),
        in_specs=[
            pl.BlockSpec((BLK, D_MODEL), lambda i: (i, 0)),
            full(M1), full(Wall), full(S2), full(K), full(Rx), full(Rh),
        ],
        out_specs=pl.BlockSpec((BLK, D_MODEL), lambda i: (i, 0)),
        out_shape=jax.ShapeDtypeStruct((T, D_MODEL), jnp.float32),
    )(x2, M1, Wall, S2, K, Rx, Rh)
    return out.reshape(Bsz, S, D)


# speedup vs baseline: 1.4478x; 1.4478x over previous
import jax
import jax.numpy as jnp
import numpy as np
from jax.experimental import pallas as pl

N_EXP = 64
D_MODEL = 8
D_FF = 16
BLK = 2048


def _moe_block(x_ref, M1_ref, G1_ref, G2_ref, Wall_ref, S2_ref, K_ref,
               Rx_ref, Rh_ref, o_ref):
    x = x_ref[...]
    t1 = jnp.dot(x, M1_ref[...], preferred_element_type=jnp.float32)
    logits = t1[:, :N_EXP]
    gs = t1[:, N_EXP:N_EXP + D_FF]
    us = t1[:, N_EXP + D_FF:N_EXP + 2 * D_FF]

    m = jnp.max(logits, axis=1, keepdims=True)
    eq = (logits == m).astype(jnp.float32)
    e1 = jnp.exp(logits - m)
    aux = (jnp.dot(eq, G1_ref[...], preferred_element_type=jnp.float32)
           + jnp.dot(e1, G2_ref[...], preferred_element_type=jnp.float32))
    oh = jnp.where(aux[:, :N_EXP] > 0.5, 0.0, eq)
    ohw = oh / aux[:, N_EXP:N_EXP + 1]

    Wt = jnp.dot(jnp.concatenate([oh, ohw], axis=1), Wall_ref[...],
                 preferred_element_type=jnp.float32)

    xt = jnp.dot(x, Rx_ref[...], preferred_element_type=jnp.float32)
    pg = Wt[:, :128] * xt
    pu = Wt[:, 128:256] * xt
    gu = jnp.dot(jnp.concatenate([pg, pu], axis=1), S2_ref[...],
                 preferred_element_type=jnp.float32)
    g = gu[:, :D_FF]
    u = gu[:, D_FF:]
    h = (g * jax.nn.sigmoid(g)) * u

    ht = jnp.dot(h, Rh_ref[...], preferred_element_type=jnp.float32)
    pd = Wt[:, 256:384] * ht
    hs = (gs * jax.nn.sigmoid(gs)) * us
    o_ref[...] = jnp.dot(jnp.concatenate([pd, hs], axis=1), K_ref[...],
                         preferred_element_type=jnp.float32)


def kernel(hidden_states, gate_weight, Wg, Wu, Wd, Wsg, Wsu, Wsd):
    Bsz, S, D = hidden_states.shape
    T = Bsz * S
    x2 = hidden_states.reshape(T, D)

    M1 = jnp.concatenate([gate_weight.T, Wsg.T, Wsu.T], axis=1)
    G1 = np.zeros((N_EXP, N_EXP + D_MODEL), np.float32)
    G1[:, :N_EXP] = np.tril(np.ones((N_EXP, N_EXP), np.float32), k=-1).T
    G2 = np.zeros((N_EXP, N_EXP + D_MODEL), np.float32)
    G2[:, N_EXP:] = 1.0
    Wall = jnp.concatenate([
        jnp.concatenate([Wg.reshape(N_EXP, 128), Wu.reshape(N_EXP, 128),
                         jnp.zeros((N_EXP, 128), jnp.float32)], axis=1),
        jnp.concatenate([jnp.zeros((N_EXP, 256), jnp.float32),
                         Wd.reshape(N_EXP, 128)], axis=1),
    ], axis=0)
    S8 = np.kron(np.eye(D_FF, dtype=np.float32), np.ones((D_MODEL, 1), np.float32))
    S16 = np.kron(np.eye(D_MODEL, dtype=np.float32), np.ones((D_FF, 1), np.float32))
    S2 = np.zeros((256, 2 * D_FF), np.float32)
    S2[:128, :D_FF] = S8
    S2[128:, D_FF:] = S8
    K = jnp.concatenate([jnp.asarray(S16), Wsd.T], axis=0)
    Rx = np.kron(np.ones((1, D_FF), np.float32), np.eye(D_MODEL, dtype=np.float32))
    Rh = np.kron(np.ones((1, D_MODEL), np.float32), np.eye(D_FF, dtype=np.float32))

    G1, G2, S2, Rx, Rh = map(jnp.asarray, (G1, G2, S2, Rx, Rh))

    full = lambda arr: pl.BlockSpec(arr.shape, lambda i: (0, 0))
    out = pl.pallas_call(
        _moe_block,
        grid=(T // BLK,),
        in_specs=[
            pl.BlockSpec((BLK, D_MODEL), lambda i: (i, 0)),
            full(M1), full(G1), full(G2), full(Wall), full(S2), full(K),
            full(Rx), full(Rh),
        ],
        out_specs=pl.BlockSpec((BLK, D_MODEL), lambda i: (i, 0)),
        out_shape=jax.ShapeDtypeStruct((T, D_MODEL), jnp.float32),
    )(x2, M1, G1, G2, Wall, S2, K, Rx, Rh)
    return out.reshape(Bsz, S, D)


# speedup vs baseline: 1.4519x; 1.0028x over previous
import jax
import jax.numpy as jnp
import numpy as np
from jax.experimental import pallas as pl
from jax.experimental.pallas import tpu as pltpu

N_EXP = 64
D_MODEL = 8
D_FF = 16
BLK = 2048


def _moe_block(x_ref, M1_ref, G1_ref, G2_ref, Wall_ref, S2_ref, K_ref,
               Rx_ref, Rh_ref, o_ref):
    x = x_ref[...]
    t1 = jnp.dot(x, M1_ref[...], preferred_element_type=jnp.float32)
    logits = t1[:, :N_EXP]
    gs = t1[:, N_EXP:N_EXP + D_FF]
    us = t1[:, N_EXP + D_FF:N_EXP + 2 * D_FF]

    m = jnp.max(logits, axis=1, keepdims=True)
    eq = (logits == m).astype(jnp.float32)
    e1 = jnp.exp(logits - m)
    aux = (jnp.dot(eq, G1_ref[...], preferred_element_type=jnp.float32)
           + jnp.dot(e1, G2_ref[...], preferred_element_type=jnp.float32))
    oh = jnp.where(aux[:, :N_EXP] > 0.5, 0.0, eq)
    ohw = oh / aux[:, N_EXP:N_EXP + 1]

    Wt = jnp.dot(jnp.concatenate([oh, ohw], axis=1), Wall_ref[...],
                 preferred_element_type=jnp.float32)

    xt = jnp.dot(x, Rx_ref[...], preferred_element_type=jnp.float32)
    pg = Wt[:, :128] * xt
    pu = Wt[:, 128:256] * xt
    gu = jnp.dot(jnp.concatenate([pg, pu], axis=1), S2_ref[...],
                 preferred_element_type=jnp.float32)
    g = gu[:, :D_FF]
    u = gu[:, D_FF:]
    h = (g * jax.nn.sigmoid(g)) * u

    ht = jnp.dot(h, Rh_ref[...], preferred_element_type=jnp.float32)
    pd = Wt[:, 256:384] * ht
    hs = (gs * jax.nn.sigmoid(gs)) * us
    o_ref[...] = jnp.dot(jnp.concatenate([pd, hs], axis=1), K_ref[...],
                         preferred_element_type=jnp.float32)


def kernel(hidden_states, gate_weight, Wg, Wu, Wd, Wsg, Wsu, Wsd):
    Bsz, S, D = hidden_states.shape
    T = Bsz * S
    x2 = hidden_states.reshape(T, D)

    M1 = jnp.concatenate([gate_weight.T, Wsg.T, Wsu.T], axis=1)
    G1 = np.zeros((N_EXP, N_EXP + D_MODEL), np.float32)
    G1[:, :N_EXP] = np.tril(np.ones((N_EXP, N_EXP), np.float32), k=-1).T
    G2 = np.zeros((N_EXP, N_EXP + D_MODEL), np.float32)
    G2[:, N_EXP:] = 1.0
    Wall = jnp.concatenate([
        jnp.concatenate([Wg.reshape(N_EXP, 128), Wu.reshape(N_EXP, 128),
                         jnp.zeros((N_EXP, 128), jnp.float32)], axis=1),
        jnp.concatenate([jnp.zeros((N_EXP, 256), jnp.float32),
                         Wd.reshape(N_EXP, 128)], axis=1),
    ], axis=0)
    S8 = np.kron(np.eye(D_FF, dtype=np.float32), np.ones((D_MODEL, 1), np.float32))
    S16 = np.kron(np.eye(D_MODEL, dtype=np.float32), np.ones((D_FF, 1), np.float32))
    S2 = np.zeros((256, 2 * D_FF), np.float32)
    S2[:128, :D_FF] = S8
    S2[128:, D_FF:] = S8
    K = jnp.concatenate([jnp.asarray(S16), Wsd.T], axis=0)
    Rx = np.kron(np.ones((1, D_FF), np.float32), np.eye(D_MODEL, dtype=np.float32))
    Rh = np.kron(np.ones((1, D_MODEL), np.float32), np.eye(D_FF, dtype=np.float32))

    G1, G2, S2, Rx, Rh = map(jnp.asarray, (G1, G2, S2, Rx, Rh))

    full = lambda arr: pl.BlockSpec(arr.shape, lambda i: (0, 0))
    out = pl.pallas_call(
        _moe_block,
        grid=(T // BLK,),
        in_specs=[
            pl.BlockSpec((BLK, D_MODEL), lambda i: (i, 0)),
            full(M1), full(G1), full(G2), full(Wall), full(S2), full(K),
            full(Rx), full(Rh),
        ],
        out_specs=pl.BlockSpec((BLK, D_MODEL), lambda i: (i, 0)),
        out_shape=jax.ShapeDtypeStruct((T, D_MODEL), jnp.float32),
        compiler_params=pltpu.CompilerParams(dimension_semantics=("parallel",)),
    )(x2, M1, G1, G2, Wall, S2, K, Rx, Rh)
    return out.reshape(Bsz, S, D)


# speedup vs baseline: 1.4874x; 1.0244x over previous
import functools
import jax
import jax.numpy as jnp
import numpy as np
from jax.experimental import pallas as pl
from jax.experimental.pallas import tpu as pltpu

N_EXP = 64
D_MODEL = 8
D_FF = 16
BLK = 2048

_DN_T = (((1,), (1,)), ((), ()))


def _moe_block(x_ref, gw_ref, Wg_ref, Wu_ref, Wd_ref, Wsg_ref, Wsu_ref,
               Wsd_ref, G1_ref, G2_ref, S8_ref, S16_ref, Rx_ref, Rh_ref,
               o_ref):
    f32 = jnp.float32
    dot = functools.partial(jnp.dot, preferred_element_type=f32)
    dot_t = functools.partial(jax.lax.dot_general, dimension_numbers=_DN_T,
                              preferred_element_type=f32)
    x = x_ref[...]
    logits = dot_t(x, gw_ref[...])
    gs = dot_t(x, Wsg_ref[...])
    us = dot_t(x, Wsu_ref[...])

    m = jnp.max(logits, axis=1, keepdims=True)
    eq = (logits == m).astype(f32)
    e1 = jnp.exp(logits - m)
    aux = dot(eq, G1_ref[...]) + dot(e1, G2_ref[...])
    oh = jnp.where(aux[:, :N_EXP] > 0.5, 0.0, eq)
    ohw = oh / aux[:, N_EXP:N_EXP + 1]

    Wtg = dot(oh, Wg_ref[...])
    Wtu = dot(oh, Wu_ref[...])
    Wtd = dot(ohw, Wd_ref[...])

    xt = dot(x, Rx_ref[...])
    g = dot(Wtg * xt, S8_ref[...])
    u = dot(Wtu * xt, S8_ref[...])
    h = (g * jax.nn.sigmoid(g)) * u

    ht = dot(h, Rh_ref[...])
    routed = dot(Wtd * ht, S16_ref[...])
    hs = (gs * jax.nn.sigmoid(gs)) * us
    o_ref[...] = routed + dot_t(hs, Wsd_ref[...])


def kernel(hidden_states, gate_weight, Wg, Wu, Wd, Wsg, Wsu, Wsd):
    Bsz, S, D = hidden_states.shape
    T = Bsz * S
    x2 = hidden_states.reshape(T, D)

    G1 = np.zeros((N_EXP, N_EXP + D_MODEL), np.float32)
    G1[:, :N_EXP] = np.tril(np.ones((N_EXP, N_EXP), np.float32), k=-1).T
    G2 = np.zeros((N_EXP, N_EXP + D_MODEL), np.float32)
    G2[:, N_EXP:] = 1.0
    S8 = np.kron(np.eye(D_FF, dtype=np.float32), np.ones((D_MODEL, 1), np.float32))
    S16 = np.kron(np.eye(D_MODEL, dtype=np.float32), np.ones((D_FF, 1), np.float32))
    Rx = np.kron(np.ones((1, D_FF), np.float32), np.eye(D_MODEL, dtype=np.float32))
    Rh = np.kron(np.ones((1, D_MODEL), np.float32), np.eye(D_FF, dtype=np.float32))
    G1, G2, S8, S16, Rx, Rh = map(jnp.asarray, (G1, G2, S8, S16, Rx, Rh))

    full = lambda arr: pl.BlockSpec(arr.shape, lambda i: (0,) * arr.ndim)
    out = pl.pallas_call(
        _moe_block,
        grid=(T // BLK,),
        in_specs=[pl.BlockSpec((BLK, D_MODEL), lambda i: (i, 0))]
        + [full(a) for a in (gate_weight, Wg.reshape(N_EXP, 128),
                             Wu.reshape(N_EXP, 128), Wd.reshape(N_EXP, 128),
                             Wsg, Wsu, Wsd, G1, G2, S8, S16, Rx, Rh)],
        out_specs=pl.BlockSpec((BLK, D_MODEL), lambda i: (i, 0)),
        out_shape=jax.ShapeDtypeStruct((T, D_MODEL), jnp.float32),
        compiler_params=pltpu.CompilerParams(dimension_semantics=("parallel",)),
    )(x2, gate_weight, Wg.reshape(N_EXP, 128), Wu.reshape(N_EXP, 128),
      Wd.reshape(N_EXP, 128), Wsg, Wsu, Wsd, G1, G2, S8, S16, Rx, Rh)
    return out.reshape(Bsz, S, D)


# speedup vs baseline: 1.7385x; 1.1689x over previous
import functools
import jax
import jax.numpy as jnp
import numpy as np
from jax.experimental import pallas as pl
from jax.experimental.pallas import tpu as pltpu

N_EXP = 64
D_MODEL = 8
D_FF = 16
BLK = 2048


def _moe_block(x_ref, M2_ref, Wg_ref, Wu_ref, Wd_ref, G1_ref, G2_ref,
               S8q_ref, S16_ref, Rx_ref, Rh2_ref, Wsd2_ref, o_ref):
    f32 = jnp.float32
    dot = functools.partial(jnp.dot, preferred_element_type=f32)
    x = x_ref[...]
    t1 = dot(x, M2_ref[...])
    logits = t1[:, :N_EXP]

    m = jnp.max(logits, axis=1, keepdims=True)
    d = logits - m
    eq = (d == 0.0).astype(f32)
    e1 = jnp.exp(d)
    aux = dot(eq, G1_ref[...]) + dot(e1, G2_ref[...])
    oh = jnp.where(aux[:, :N_EXP] > 0.5, 0.0, eq)
    ohw = oh / aux[:, N_EXP:N_EXP + 1]

    Wtg = dot(oh, Wg_ref[...])
    Wtu = dot(oh, Wu_ref[...])
    Wtd = dot(ohw, Wd_ref[...])

    xt = dot(x, Rx_ref[...])
    q = dot(Wtg * xt, S8q_ref[...]) + t1[:, N_EXP:N_EXP + 32]
    r = dot(Wtu * xt, S8q_ref[...]) + t1[:, N_EXP + 32:]
    hh = (q * jax.nn.sigmoid(q)) * r

    ht = dot(hh, Rh2_ref[...])
    routed = dot(Wtd * ht, S16_ref[...])
    o_ref[...] = routed + dot(hh, Wsd2_ref[...])


def kernel(hidden_states, gate_weight, Wg, Wu, Wd, Wsg, Wsu, Wsd):
    Bsz, S, D = hidden_states.shape
    T = Bsz * S
    x2 = hidden_states.reshape(T, D)

    Z16 = jnp.zeros((D_MODEL, D_FF), jnp.float32)
    M2 = jnp.concatenate([gate_weight.T, Z16, Wsg.T, Z16, Wsu.T], axis=1)
    G1 = np.zeros((N_EXP, N_EXP + D_MODEL), np.float32)
    G1[:, :N_EXP] = np.tril(np.ones((N_EXP, N_EXP), np.float32), k=-1).T
    G2 = np.zeros((N_EXP, N_EXP + D_MODEL), np.float32)
    G2[:, N_EXP:] = 1.0
    S8 = np.kron(np.eye(D_FF, dtype=np.float32), np.ones((D_MODEL, 1), np.float32))
    S8q = np.zeros((128, 32), np.float32)
    S8q[:, :D_FF] = S8
    S16 = np.kron(np.eye(D_MODEL, dtype=np.float32), np.ones((D_FF, 1), np.float32))
    Rx = np.kron(np.ones((1, D_FF), np.float32), np.eye(D_MODEL, dtype=np.float32))
    Rh2 = np.zeros((32, 128), np.float32)
    Rh2[:D_FF] = np.kron(np.ones((1, D_MODEL), np.float32),
                         np.eye(D_FF, dtype=np.float32))
    G1, G2, S8q, S16, Rx, Rh2 = map(jnp.asarray, (G1, G2, S8q, S16, Rx, Rh2))
    Wsd2 = jnp.concatenate([jnp.zeros((D_FF, D_MODEL), jnp.float32), Wsd.T],
                           axis=0)

    full = lambda arr: pl.BlockSpec(arr.shape, lambda i: (0,) * arr.ndim)
    args = (x2, M2, Wg.reshape(N_EXP, 128), Wu.reshape(N_EXP, 128),
            Wd.reshape(N_EXP, 128), G1, G2, S8q, S16, Rx, Rh2, Wsd2)
    out = pl.pallas_call(
        _moe_block,
        grid=(T // BLK,),
        in_specs=[pl.BlockSpec((BLK, D_MODEL), lambda i: (i, 0))]
        + [full(a) for a in args[1:]],
        out_specs=pl.BlockSpec((BLK, D_MODEL), lambda i: (i, 0)),
        out_shape=jax.ShapeDtypeStruct((T, D_MODEL), jnp.float32),
        compiler_params=pltpu.CompilerParams(dimension_semantics=("parallel",)),
    )(*args)
    return out.reshape(Bsz, S, D)
